# baseline (speedup 1.0000x reference)
import functools

import jax
import jax.numpy as jnp
from jax import lax
from jax.experimental import pallas as pl
from jax.experimental.pallas import tpu as pltpu
from jax.experimental.pallas import tpu_sc as plsc

_DECAY = 0.0001

_B = 16384
_D = 64
_NI = 1000000
_NU = 100000
_NW = 32
_BPW = _B // _NW

_ET_I = 248
_ET_U = 28
_SW = 512
_NB_I = _ET_I * 128 // _SW
_NB_U = _ET_U * 128 // _SW
_PAD_I = 1000064
_PAD_U = 100096
_CAP_I = 40
_CAP_U = 176
_ICH = 4096

_mesh = plsc.VectorSubcoreMesh(core_axis_name="c", subcore_axis_name="s")

_i32 = jnp.int32


def _iota16():
    return lax.iota(_i32, 16)


@functools.partial(
    pl.kernel,
    mesh=_mesh,
    compiler_params=pltpu.CompilerParams(needs_layout_passes=False),
    out_type=(
        jax.ShapeDtypeStruct(((_B + 1) * _D,), jnp.float32),
        jax.ShapeDtypeStruct(((_B + 1) * _D,), jnp.float32),
        jax.ShapeDtypeStruct(((_B + 1) * _D,), jnp.float32),
    ),
    scratch_types=[
        pltpu.VMEM((_ICH,), _i32),
        pltpu.VMEM((64,), _i32),
        pltpu.VMEM((64,), _i32),
        pltpu.VMEM((16,), _i32),
        pltpu.VMEM((_NB_I * _CAP_I,), _i32),
        pltpu.VMEM((_NB_I * _CAP_I,), _i32),
        pltpu.VMEM((_NB_U * _CAP_U,), _i32),
        pltpu.VMEM((_D, _SW), jnp.float32),
        pltpu.VMEM((_D, _SW), jnp.float32),
        pltpu.VMEM((256, _D), jnp.float32),
        pltpu.VMEM((16 * _D,), jnp.float32),
        pltpu.SemaphoreType.DMA,
        pltpu.SemaphoreType.DMA,
        pltpu.SemaphoreType.DMA,
    ],
)
def _sc_gather(users_t, items_t, users, pos_items, neg_items,
               gu, gp, gn,
               idx_buf, cp, cn, cu, bp, bn, bu,
               slab_a, slab_b, pack, drain, sem_a, sem_b, sem_s):
    wid = lax.axis_index("s") * 2 + lax.axis_index("c")
    it_lo = wid * _ET_I * 128
    us_lo = wid * _ET_U * 128

    z16 = jnp.zeros((16,), _i32)
    for q in range(4):
        cp[pl.ds(q * 16, 16)] = z16
        cn[pl.ds(q * 16, 16)] = z16
    cu[pl.ds(0, 16)] = z16

    def scan_array(arr, cnt_ref, buf_ref, lo_ent, n_ent, cap):
        def chunk_body(c, carry):
            pltpu.sync_copy(arr.at[pl.ds(c * _ICH, _ICH)], idx_buf)

            def vreg_body(g, carry2):
                e = idx_buf[pl.ds(g * 16, 16)]
                slot = c * _ICH + g * 16 + _iota16()
                el = e - lo_ent
                m = (e >= lo_ent) & (e < lo_ent + n_ent)
                b = jnp.where(m, el >> 9, 0)
                rank, last = plsc.scan_count(b, m)
                cb = plsc.load_gather(cnt_ref, [b], mask=m)
                pos_ = b * cap + jnp.minimum(cb + rank - 1, cap - 1)
                pay = el * 16384 + slot
                plsc.store_scatter(buf_ref, [pos_], pay, mask=m)
                plsc.store_scatter(cnt_ref, [b], cb + rank, mask=m & last)
                return carry2

            lax.fori_loop(0, _ICH // 16, vreg_body, 0)
            return carry

        lax.fori_loop(0, _B // _ICH, chunk_body, 0)

    scan_array(pos_items, cp, bp, it_lo, _ET_I * 128, _CAP_I)
    scan_array(neg_items, cn, bn, it_lo, _ET_I * 128, _CAP_I)
    scan_array(users, cu, bu, us_lo, _ET_U * 128, _CAP_U)

    def slab_start(lo_ent, s, pad_end):
        return jnp.minimum(lo_ent + s * _SW, pad_end - _SW)

    def extract(slab, start_rel, cnt_ref, buf_ref, s, cap, out_flat, issued0):
        kv = plsc.load_gather(cnt_ref, [jnp.full((16,), s, _i32)])
        k = jnp.minimum(kv[0], cap)

        def group_body(g, issued):
            @pl.when(issued >= 15)
            def _():
                pltpu.make_async_copy(out_flat.at[pl.ds(0, 16 * _D)], drain,
                                      sem_s).wait()

            ring = (issued & 15) * 16
            pay = buf_ref[pl.ds(s * cap + g * 16, 16)]
            valid = (g * 16 + _iota16()) < k
            el = pay >> 14
            lane = jnp.where(valid, el - start_rel, 0)
            slot = jnp.where(valid, pay & 16383, _B)
            rows = ring + _iota16()
            for f in range(_D):
                vals = plsc.load_gather(slab, [jnp.full((16,), f, _i32), lane],
                                        mask=valid)
                plsc.store_scatter(pack, [rows, jnp.full((16,), f, _i32)],
                                   vals, mask=valid)
            for j in range(16):
                sl_ = slot[j]
                pltpu.async_copy(
                    pack.at[ring + j],
                    out_flat.at[pl.ds(pl.multiple_of(sl_ * _D, _D), _D)],
                    sem_s)
            return issued + 1

        return lax.fori_loop(0, (k + 15) // 16, group_body, issued0)

    def fire(table, lo_ent, s, pad_end, slab, sem):
        off = pl.multiple_of(slab_start(lo_ent, s, pad_end), 128)
        pltpu.async_copy(table.at[:, pl.ds(off, _SW)], slab, sem)

    def wait_slab(table, slab, sem):
        pltpu.make_async_copy(table.at[:, pl.ds(0, _SW)], slab, sem).wait()

    def run_table(table, lo_ent, n_slab, pad_end, jobs, issued0):
        fire(table, lo_ent, 0, pad_end, slab_a, sem_a)

        def slab_body(s, total):
            def on(slab, sem, other_slab, other_sem):
                res = total

                @pl.when(s + 1 < n_slab)
                def _():
                    fire(table, lo_ent, s + 1, pad_end, other_slab, other_sem)

                wait_slab(table, slab, sem)
                start_rel = slab_start(lo_ent, s, pad_end) - lo_ent
                for cnt_ref, buf_ref, cap, out_flat in jobs:
                    res = extract(slab, start_rel, cnt_ref, buf_ref, s, cap,
                                  out_flat, res)
                return res

            return lax.cond(s % 2 == 0,
                            lambda: on(slab_a, sem_a, slab_b, sem_b),
                            lambda: on(slab_b, sem_b, slab_a, sem_a))

        return lax.fori_loop(0, n_slab, slab_body, issued0)

    issued = run_table(items_t, it_lo, _NB_I, _PAD_I,
                       [(cp, bp, _CAP_I, gp), (cn, bn, _CAP_I, gn)],
                       jnp.int32(0))
    issued = run_table(users_t, us_lo, _NB_U, _PAD_U,
                       [(cu, bu, _CAP_U, gu)], issued)

    def drain_body(i, carry):
        pltpu.make_async_copy(gu.at[pl.ds(0, 16 * _D)], drain, sem_s).wait()
        return carry

    lax.fori_loop(0, jnp.minimum(issued, 15), drain_body, 0)


@functools.partial(
    pl.kernel,
    mesh=_mesh,
    compiler_params=pltpu.CompilerParams(needs_layout_passes=False),
    out_type=jax.ShapeDtypeStruct((_B, 16), jnp.float32),
    scratch_types=[
        pltpu.VMEM((128 * _D,), jnp.float32),
        pltpu.VMEM((128 * _D,), jnp.float32),
        pltpu.VMEM((128 * _D,), jnp.float32),
        pltpu.VMEM((_BPW, 16), jnp.float32),
    ],
)
def _sc_scores(gu, gp, gn, out_hbm, ub, pb, nb, part):
    wid = lax.axis_index("s") * 2 + lax.axis_index("c")
    base = wid * _BPW

    for h in range(_BPW // 128):
        off = (base + h * 128) * _D
        pltpu.sync_copy(gu.at[pl.ds(off, 128 * _D)], ub)
        pltpu.sync_copy(gp.at[pl.ds(off, 128 * _D)], pb)
        pltpu.sync_copy(gn.at[pl.ds(off, 128 * _D)], nb)

        def row_body(i, carry):
            t = jnp.zeros((16,), jnp.float32)
            for c in range(_D // 16):
                sl = pl.ds(i * _D + c * 16, 16)
                t = t + ub[sl] * (pb[sl] - nb[sl])
            part[h * 128 + i, :] = t
            return carry

        lax.fori_loop(0, 128, row_body, 0)

    pltpu.sync_copy(part, out_hbm.at[pl.ds(base, _BPW)])


def _tc_loss_body(part_ref, cu_ref, cp_ref, cn_ref, out_ref):
    diff = jnp.sum(part_ref[...], axis=1)
    ls = jnp.minimum(diff, 0.0) - jnp.log1p(jnp.exp(-jnp.abs(diff)))
    loss = -jnp.sum(ls) / _B
    reg = 0.5 * (jnp.sum(cu_ref[...] * cu_ref[...])
                 + jnp.sum(cp_ref[...] * cp_ref[...])
                 + jnp.sum(cn_ref[...] * cn_ref[...])) / _B
    out_ref[...] = jnp.broadcast_to(loss + _DECAY * reg, (1, 1))


def kernel(users_emb, items_emb, users, pos_items, neg_items,
           current_user_emb, current_pos_item_emb, current_neg_item_emb):
    gu, gp, gn = _sc_gather(users_emb.T, items_emb.T,
                            users.astype(_i32),
                            pos_items.astype(_i32),
                            neg_items.astype(_i32))
    part = _sc_scores(gu, gp, gn)
    out = pl.pallas_call(
        _tc_loss_body,
        out_shape=jax.ShapeDtypeStruct((1, 1), jnp.float32),
    )(part, current_user_emb.T, current_pos_item_emb.T, current_neg_item_emb.T)
    return out[0, 0]


# speedup vs baseline: 2.1406x; 2.1406x over previous
import functools

import jax
import jax.numpy as jnp
from jax import lax
from jax.experimental import pallas as pl
from jax.experimental.pallas import tpu as pltpu
from jax.experimental.pallas import tpu_sc as plsc

_DECAY = 0.0001

_B = 16384
_D = 64
_NI = 1000000
_NU = 100000
_NW = 32
_BPW = _B // _NW

_ET_I = 248
_ET_U = 28
_SW = 512
_NB_I = _ET_I * 128 // _SW
_NB_U = _ET_U * 128 // _SW
_PAD_I = 1000064
_PAD_U = 100096
_CAP_I = 40
_CAP_U = 176
_ICH = 4096

_mesh = plsc.VectorSubcoreMesh(core_axis_name="c", subcore_axis_name="s")

_i32 = jnp.int32


def _iota16():
    return lax.iota(_i32, 16)


@functools.partial(
    pl.kernel,
    mesh=_mesh,
    compiler_params=pltpu.CompilerParams(needs_layout_passes=False),
    out_type=(
        jax.ShapeDtypeStruct(((_B + _NW) * _D,), jnp.float32),
        jax.ShapeDtypeStruct(((_B + _NW) * _D,), jnp.float32),
        jax.ShapeDtypeStruct(((_B + _NW) * _D,), jnp.float32),
    ),
    scratch_types=[
        pltpu.VMEM((_ICH,), _i32),
        pltpu.VMEM((64,), _i32),
        pltpu.VMEM((64,), _i32),
        pltpu.VMEM((16,), _i32),
        pltpu.VMEM((_NB_I * _CAP_I,), _i32),
        pltpu.VMEM((_NB_I * _CAP_I,), _i32),
        pltpu.VMEM((_NB_U * _CAP_U,), _i32),
        pltpu.VMEM((_D, _SW), jnp.float32),
        pltpu.VMEM((_D, _SW), jnp.float32),
        pltpu.VMEM((256, _D), jnp.float32),
        pltpu.VMEM((16 * _D,), jnp.float32),
        pltpu.SemaphoreType.DMA,
        pltpu.SemaphoreType.DMA,
        pltpu.SemaphoreType.DMA,
    ],
)
def _sc_gather(users_t, items_t, users, pos_items, neg_items,
               gu, gp, gn,
               idx_buf, cp, cn, cu, bp, bn, bu,
               slab_a, slab_b, pack, drain, sem_a, sem_b, sem_s):
    wid = lax.axis_index("s") * 2 + lax.axis_index("c")
    it_lo = wid * _ET_I * 128
    us_lo = wid * _ET_U * 128

    z16 = jnp.zeros((16,), _i32)
    for q in range(4):
        cp[pl.ds(q * 16, 16)] = z16
        cn[pl.ds(q * 16, 16)] = z16
    cu[pl.ds(0, 16)] = z16

    def scan_array(arr, cnt_ref, buf_ref, lo_ent, n_ent, cap):
        def chunk_body(c, carry):
            pltpu.sync_copy(arr.at[pl.ds(c * _ICH, _ICH)], idx_buf)

            def vreg_body(g, carry2):
                e = idx_buf[pl.ds(g * 16, 16)]
                slot = c * _ICH + g * 16 + _iota16()
                el = e - lo_ent
                m = (e >= lo_ent) & (e < lo_ent + n_ent)
                b = jnp.where(m, el >> 9, 0)
                rank, last = plsc.scan_count(b, m)
                cb = plsc.load_gather(cnt_ref, [b], mask=m)
                pos_ = b * cap + jnp.minimum(cb + rank - 1, cap - 1)
                pay = el * 16384 + slot
                plsc.store_scatter(buf_ref, [pos_], pay, mask=m)
                plsc.store_scatter(cnt_ref, [b], cb + rank, mask=m & last)
                return carry2

            lax.fori_loop(0, _ICH // 16, vreg_body, 0)
            return carry

        lax.fori_loop(0, _B // _ICH, chunk_body, 0)

    scan_array(pos_items, cp, bp, it_lo, _ET_I * 128, _CAP_I)
    scan_array(neg_items, cn, bn, it_lo, _ET_I * 128, _CAP_I)
    scan_array(users, cu, bu, us_lo, _ET_U * 128, _CAP_U)

    def slab_start(lo_ent, s, pad_end):
        return jnp.minimum(lo_ent + s * _SW, pad_end - _SW)

    def extract(slab, start_rel, cnt_ref, buf_ref, s, cap, out_flat, issued0):
        kv = plsc.load_gather(cnt_ref, [jnp.full((16,), s, _i32)])
        k = jnp.minimum(kv[0], cap)

        def group_body(g, issued):
            @pl.when(issued >= 15)
            def _():
                pltpu.make_async_copy(out_flat.at[pl.ds(0, 16 * _D)], drain,
                                      sem_s).wait()

            ring = (issued & 15) * 16
            pay = buf_ref[pl.ds(s * cap + g * 16, 16)]
            valid = (g * 16 + _iota16()) < k
            el = pay >> 14
            lane = jnp.where(valid, el - start_rel, 0)
            slot = jnp.where(valid, pay & 16383, _B + wid)
            rows = ring + _iota16()
            for f in range(_D):
                vals = plsc.load_gather(slab, [jnp.full((16,), f, _i32), lane],
                                        mask=valid)
                plsc.store_scatter(pack, [rows, jnp.full((16,), f, _i32)],
                                   vals, mask=valid)
            for j in range(16):
                sl_ = slot[j]
                pltpu.async_copy(
                    pack.at[ring + j],
                    out_flat.at[pl.ds(pl.multiple_of(sl_ * _D, _D), _D)],
                    sem_s)
            return issued + 1

        return lax.fori_loop(0, (k + 15) // 16, group_body, issued0)

    def fire(table, lo_ent, s, pad_end, slab, sem):
        off = pl.multiple_of(slab_start(lo_ent, s, pad_end), 128)
        pltpu.async_copy(table.at[:, pl.ds(off, _SW)], slab, sem)

    def wait_slab(table, slab, sem):
        pltpu.make_async_copy(table.at[:, pl.ds(0, _SW)], slab, sem).wait()

    def run_table(table, lo_ent, n_slab, pad_end, jobs, issued0):
        fire(table, lo_ent, 0, pad_end, slab_a, sem_a)

        def slab_body(s, total):
            def on(slab, sem, other_slab, other_sem):
                res = total

                @pl.when(s + 1 < n_slab)
                def _():
                    fire(table, lo_ent, s + 1, pad_end, other_slab, other_sem)

                wait_slab(table, slab, sem)
                start_rel = slab_start(lo_ent, s, pad_end) - lo_ent
                for cnt_ref, buf_ref, cap, out_flat in jobs:
                    res = extract(slab, start_rel, cnt_ref, buf_ref, s, cap,
                                  out_flat, res)
                return res

            return lax.cond(s % 2 == 0,
                            lambda: on(slab_a, sem_a, slab_b, sem_b),
                            lambda: on(slab_b, sem_b, slab_a, sem_a))

        return lax.fori_loop(0, n_slab, slab_body, issued0)

    issued = run_table(items_t, it_lo, _NB_I, _PAD_I,
                       [(cp, bp, _CAP_I, gp), (cn, bn, _CAP_I, gn)],
                       jnp.int32(0))
    issued = run_table(users_t, us_lo, _NB_U, _PAD_U,
                       [(cu, bu, _CAP_U, gu)], issued)

    def drain_body(i, carry):
        pltpu.make_async_copy(gu.at[pl.ds(0, 16 * _D)], drain, sem_s).wait()
        return carry

    lax.fori_loop(0, jnp.minimum(issued, 15), drain_body, 0)


@functools.partial(
    pl.kernel,
    mesh=_mesh,
    compiler_params=pltpu.CompilerParams(needs_layout_passes=False),
    out_type=jax.ShapeDtypeStruct((_B, 16), jnp.float32),
    scratch_types=[
        pltpu.VMEM((128 * _D,), jnp.float32),
        pltpu.VMEM((128 * _D,), jnp.float32),
        pltpu.VMEM((128 * _D,), jnp.float32),
        pltpu.VMEM((_BPW, 16), jnp.float32),
    ],
)
def _sc_scores(gu, gp, gn, out_hbm, ub, pb, nb, part):
    wid = lax.axis_index("s") * 2 + lax.axis_index("c")
    base = wid * _BPW

    for h in range(_BPW // 128):
        off = (base + h * 128) * _D
        pltpu.sync_copy(gu.at[pl.ds(off, 128 * _D)], ub)
        pltpu.sync_copy(gp.at[pl.ds(off, 128 * _D)], pb)
        pltpu.sync_copy(gn.at[pl.ds(off, 128 * _D)], nb)

        def row_body(i, carry):
            t = jnp.zeros((16,), jnp.float32)
            for c in range(_D // 16):
                sl = pl.ds(i * _D + c * 16, 16)
                t = t + ub[sl] * (pb[sl] - nb[sl])
            part[h * 128 + i, :] = t
            return carry

        lax.fori_loop(0, 128, row_body, 0)

    pltpu.sync_copy(part, out_hbm.at[pl.ds(base, _BPW)])


def _tc_loss_body(part_ref, cu_ref, cp_ref, cn_ref, out_ref):
    diff = jnp.sum(part_ref[...], axis=1)
    ls = jnp.minimum(diff, 0.0) - jnp.log1p(jnp.exp(-jnp.abs(diff)))
    loss = -jnp.sum(ls) / _B
    reg = 0.5 * (jnp.sum(cu_ref[...] * cu_ref[...])
                 + jnp.sum(cp_ref[...] * cp_ref[...])
                 + jnp.sum(cn_ref[...] * cn_ref[...])) / _B
    out_ref[...] = jnp.broadcast_to(loss + _DECAY * reg, (1, 1))


def kernel(users_emb, items_emb, users, pos_items, neg_items,
           current_user_emb, current_pos_item_emb, current_neg_item_emb):
    gu, gp, gn = _sc_gather(users_emb.T, items_emb.T,
                            users.astype(_i32),
                            pos_items.astype(_i32),
                            neg_items.astype(_i32))
    part = _sc_scores(gu, gp, gn)
    out = pl.pallas_call(
        _tc_loss_body,
        out_shape=jax.ShapeDtypeStruct((1, 1), jnp.float32),
    )(part, current_user_emb.T, current_pos_item_emb.T, current_neg_item_emb.T)
    return out[0, 0]


# speedup vs baseline: 2.7954x; 1.3059x over previous
import functools

import jax
import jax.numpy as jnp
from jax import lax
from jax.experimental import pallas as pl
from jax.experimental.pallas import tpu as pltpu
from jax.experimental.pallas import tpu_sc as plsc

_DECAY = 0.0001

_B = 16384
_D = 64
_NI = 1000000
_NU = 100000
_NW = 32
_BPW = _B // _NW

_RI = 32768
_RU = 4096
_SW = 512
_NB_I = _RI // _SW
_NB_U = _RU // _SW
_CAP0 = 48
_CELL = _NW * _CAP0
_PAD_I = 1000064
_PAD_U = 100096
_CAP_I = 40
_CAP_U = 176
_ICH = 4096

_mesh = plsc.VectorSubcoreMesh(core_axis_name="c", subcore_axis_name="s")

_i32 = jnp.int32


def _iota16():
    return lax.iota(_i32, 16)


@functools.partial(
    pl.kernel,
    mesh=_mesh,
    compiler_params=pltpu.CompilerParams(needs_layout_passes=False),
    out_type=(
        jax.ShapeDtypeStruct((_NW * _CELL,), _i32),
        jax.ShapeDtypeStruct((_NW * _CELL,), _i32),
        jax.ShapeDtypeStruct((_NW * _CELL,), _i32),
        jax.ShapeDtypeStruct((_NW * _NW,), _i32),
        jax.ShapeDtypeStruct((_NW * _NW,), _i32),
        jax.ShapeDtypeStruct((_NW * _NW,), _i32),
    ),
    scratch_types=[
        pltpu.VMEM((_B // _NW,), _i32),
        pltpu.VMEM((_CELL,), _i32),
        pltpu.VMEM((_NW,), _i32),
    ],
)
def _sc_route(users, pos_items, neg_items, ru, rp, rn, cru, crp, crn,
              idxb, cells, cnts):
    wid = lax.axis_index("s") * 2 + lax.axis_index("c")
    nsl = _B // _NW

    for arr, out_cells, out_cnts, shift, mask in (
            (pos_items, rp, crp, 15, _RI - 1),
            (neg_items, rn, crn, 15, _RI - 1),
            (users, ru, cru, 12, _RU - 1)):
        pltpu.sync_copy(arr.at[pl.ds(wid * nsl, nsl)], idxb)
        cnts[pl.ds(0, 16)] = jnp.zeros((16,), _i32)
        cnts[pl.ds(16, 16)] = jnp.zeros((16,), _i32)

        def vreg_body(g, carry):
            e = idxb[pl.ds(g * 16, 16)]
            slot = wid * nsl + g * 16 + _iota16()
            d = e >> shift
            rank, last = plsc.scan_count(d, None)
            cb = plsc.load_gather(cnts, [d])
            pos_ = d * _CAP0 + jnp.minimum(cb + rank - 1, _CAP0 - 1)
            pay = (e & mask) * 16384 + slot
            plsc.store_scatter(cells, [pos_], pay)
            plsc.store_scatter(cnts, [d], cb + rank, mask=last)
            return carry

        lax.fori_loop(0, nsl // 16, vreg_body, 0)
        pltpu.sync_copy(cells, out_cells.at[pl.ds(wid * _CELL, _CELL)])
        pltpu.sync_copy(cnts, out_cnts.at[pl.ds(wid * _NW, _NW)])


@functools.partial(
    pl.kernel,
    mesh=_mesh,
    compiler_params=pltpu.CompilerParams(needs_layout_passes=False),
    out_type=(
        jax.ShapeDtypeStruct(((_B + _NW) * _D,), jnp.float32),
        jax.ShapeDtypeStruct(((_B + _NW) * _D,), jnp.float32),
        jax.ShapeDtypeStruct(((_B + _NW) * _D,), jnp.float32),
    ),
    scratch_types=[
        pltpu.VMEM((_CELL,), _i32),
        pltpu.VMEM((_NW * _NW,), _i32),
        pltpu.VMEM((64,), _i32),
        pltpu.VMEM((64,), _i32),
        pltpu.VMEM((16,), _i32),
        pltpu.VMEM((_NB_I * _CAP_I,), _i32),
        pltpu.VMEM((_NB_I * _CAP_I,), _i32),
        pltpu.VMEM((_NB_U * _CAP_U,), _i32),
        pltpu.VMEM((_D, _SW), jnp.float32),
        pltpu.VMEM((_D, _SW), jnp.float32),
        pltpu.VMEM((256, _D), jnp.float32),
        pltpu.VMEM((16 * _D,), jnp.float32),
        pltpu.SemaphoreType.DMA,
        pltpu.SemaphoreType.DMA,
        pltpu.SemaphoreType.DMA,
    ],
)
def _sc_gather(users_t, items_t, ru, rp, rn, cru, crp, crn,
               gu, gp, gn,
               inc, rcnt, cp, cn, cu, bp, bn, bu,
               slab_a, slab_b, pack, drain, sem_a, sem_b, sem_s):
    wid = lax.axis_index("s") * 2 + lax.axis_index("c")
    it_lo = wid * _RI
    us_lo = wid * _RU

    z16 = jnp.zeros((16,), _i32)
    for q in range(4):
        cp[pl.ds(q * 16, 16)] = z16
        cn[pl.ds(q * 16, 16)] = z16
    cu[pl.ds(0, 16)] = z16

    def rebucket(cells_hbm, cnts_hbm, cnt_ref, buf_ref, cap):
        pltpu.sync_copy(cnts_hbm, rcnt)
        copies = []
        for s in range(_NW):
            copies.append(pltpu.async_copy(
                cells_hbm.at[pl.ds(s * _CELL + wid * _CAP0, _CAP0)],
                inc.at[pl.ds(s * _CAP0, _CAP0)], sem_a))
        for c in copies:
            c.wait()

        def src_body(s, carry):
            kv = plsc.load_gather(rcnt, [jnp.zeros((16,), _i32) + (s * _NW + wid)])
            kc = jnp.minimum(kv[0], _CAP0)

            def grp(g, carry2):
                pay = inc[pl.ds(s * _CAP0 + g * 16, 16)]
                m = (g * 16 + _iota16()) < kc
                el = pay >> 14
                b = jnp.where(m, el >> 9, 0)
                rank, last = plsc.scan_count(b, m)
                cb = plsc.load_gather(cnt_ref, [b], mask=m)
                pos_ = b * cap + jnp.minimum(cb + rank - 1, cap - 1)
                plsc.store_scatter(buf_ref, [pos_], pay, mask=m)
                plsc.store_scatter(cnt_ref, [b], cb + rank, mask=m & last)
                return carry2

            lax.fori_loop(0, (kc + 15) // 16, grp, 0)
            return carry

        lax.fori_loop(0, _NW, src_body, 0)

    rebucket(rp, crp, cp, bp, _CAP_I)
    rebucket(rn, crn, cn, bn, _CAP_I)
    rebucket(ru, cru, cu, bu, _CAP_U)

    def slab_start(lo_ent, s, pad_end):
        return jnp.minimum(lo_ent + s * _SW, pad_end - _SW)

    def extract(slab, start_rel, cnt_ref, buf_ref, s, cap, out_flat, issued0):
        kv = plsc.load_gather(cnt_ref, [jnp.full((16,), s, _i32)])
        k = jnp.minimum(kv[0], cap)

        def group_body(g, issued):
            @pl.when(issued >= 15)
            def _():
                pltpu.make_async_copy(out_flat.at[pl.ds(0, 16 * _D)], drain,
                                      sem_s).wait()

            ring = (issued & 15) * 16
            pay = buf_ref[pl.ds(s * cap + g * 16, 16)]
            valid = (g * 16 + _iota16()) < k
            el = pay >> 14
            lane = jnp.where(valid, el - start_rel, 0)
            slot = jnp.where(valid, pay & 16383, _B + wid)
            rows = ring + _iota16()
            for f in range(_D):
                vals = plsc.load_gather(slab, [jnp.full((16,), f, _i32), lane],
                                        mask=valid)
                plsc.store_scatter(pack, [rows, jnp.full((16,), f, _i32)],
                                   vals, mask=valid)
            for j in range(16):
                sl_ = slot[j]
                pltpu.async_copy(
                    pack.at[ring + j],
                    out_flat.at[pl.ds(pl.multiple_of(sl_ * _D, _D), _D)],
                    sem_s)
            return issued + 1

        return lax.fori_loop(0, (k + 15) // 16, group_body, issued0)

    def fire(table, lo_ent, s, pad_end, slab, sem):
        off = pl.multiple_of(slab_start(lo_ent, s, pad_end), 128)
        pltpu.async_copy(table.at[:, pl.ds(off, _SW)], slab, sem)

    def wait_slab(table, slab, sem):
        pltpu.make_async_copy(table.at[:, pl.ds(0, _SW)], slab, sem).wait()

    def run_table(table, lo_ent, n_slab, pad_end, jobs, issued0):
        fire(table, lo_ent, 0, pad_end, slab_a, sem_a)

        def slab_body(s, total):
            def on(slab, sem, other_slab, other_sem):
                res = total

                @pl.when(s + 1 < n_slab)
                def _():
                    fire(table, lo_ent, s + 1, pad_end, other_slab, other_sem)

                wait_slab(table, slab, sem)
                start_rel = slab_start(lo_ent, s, pad_end) - lo_ent
                for cnt_ref, buf_ref, cap, out_flat in jobs:
                    res = extract(slab, start_rel, cnt_ref, buf_ref, s, cap,
                                  out_flat, res)
                return res

            return lax.cond(s % 2 == 0,
                            lambda: on(slab_a, sem_a, slab_b, sem_b),
                            lambda: on(slab_b, sem_b, slab_a, sem_a))

        return lax.fori_loop(0, n_slab, slab_body, issued0)

    issued = run_table(items_t, it_lo, _NB_I, _PAD_I,
                       [(cp, bp, _CAP_I, gp), (cn, bn, _CAP_I, gn)],
                       jnp.int32(0))
    issued = run_table(users_t, us_lo, _NB_U, _PAD_U,
                       [(cu, bu, _CAP_U, gu)], issued)

    def drain_body(i, carry):
        pltpu.make_async_copy(gu.at[pl.ds(0, 16 * _D)], drain, sem_s).wait()
        return carry

    lax.fori_loop(0, jnp.minimum(issued, 15), drain_body, 0)


@functools.partial(
    pl.kernel,
    mesh=_mesh,
    compiler_params=pltpu.CompilerParams(needs_layout_passes=False),
    out_type=jax.ShapeDtypeStruct((_B, 16), jnp.float32),
    scratch_types=[
        pltpu.VMEM((128 * _D,), jnp.float32),
        pltpu.VMEM((128 * _D,), jnp.float32),
        pltpu.VMEM((128 * _D,), jnp.float32),
        pltpu.VMEM((_BPW, 16), jnp.float32),
    ],
)
def _sc_scores(gu, gp, gn, out_hbm, ub, pb, nb, part):
    wid = lax.axis_index("s") * 2 + lax.axis_index("c")
    base = wid * _BPW

    for h in range(_BPW // 128):
        off = (base + h * 128) * _D
        pltpu.sync_copy(gu.at[pl.ds(off, 128 * _D)], ub)
        pltpu.sync_copy(gp.at[pl.ds(off, 128 * _D)], pb)
        pltpu.sync_copy(gn.at[pl.ds(off, 128 * _D)], nb)

        def row_body(i, carry):
            t = jnp.zeros((16,), jnp.float32)
            for c in range(_D // 16):
                sl = pl.ds(i * _D + c * 16, 16)
                t = t + ub[sl] * (pb[sl] - nb[sl])
            part[h * 128 + i, :] = t
            return carry

        lax.fori_loop(0, 128, row_body, 0)

    pltpu.sync_copy(part, out_hbm.at[pl.ds(base, _BPW)])


def _tc_loss_body(part_ref, cu_ref, cp_ref, cn_ref, out_ref):
    diff = jnp.sum(part_ref[...], axis=1)
    ls = jnp.minimum(diff, 0.0) - jnp.log1p(jnp.exp(-jnp.abs(diff)))
    loss = -jnp.sum(ls) / _B
    reg = 0.5 * (jnp.sum(cu_ref[...] * cu_ref[...])
                 + jnp.sum(cp_ref[...] * cp_ref[...])
                 + jnp.sum(cn_ref[...] * cn_ref[...])) / _B
    out_ref[...] = jnp.broadcast_to(loss + _DECAY * reg, (1, 1))


def kernel(users_emb, items_emb, users, pos_items, neg_items,
           current_user_emb, current_pos_item_emb, current_neg_item_emb):
    ru, rp, rn, cru, crp, crn = _sc_route(users.astype(_i32),
                                          pos_items.astype(_i32),
                                          neg_items.astype(_i32))
    gu, gp, gn = _sc_gather(users_emb.T, items_emb.T,
                            ru, rp, rn, cru, crp, crn)
    part = _sc_scores(gu, gp, gn)
    out = pl.pallas_call(
        _tc_loss_body,
        out_shape=jax.ShapeDtypeStruct((1, 1), jnp.float32),
    )(part, current_user_emb.T, current_pos_item_emb.T, current_neg_item_emb.T)
    return out[0, 0]


# speedup vs baseline: 2.8439x; 1.0173x over previous
import functools

import jax
import jax.numpy as jnp
from jax import lax
from jax.experimental import pallas as pl
from jax.experimental.pallas import tpu as pltpu
from jax.experimental.pallas import tpu_sc as plsc

_DECAY = 0.0001

_B = 16384
_D = 64
_NI = 1000000
_NU = 100000
_NW = 32
_BPW = _B // _NW

_RI = 32768
_RU = 4096
_SW = 512
_NB_I = _RI // _SW
_NB_U = _RU // _SW
_CAP0 = 48
_CELL = _NW * _CAP0
_PAD_I = 1000064
_PAD_U = 100096
_CAP_I = 40
_CAP_U = 176

_mesh = plsc.VectorSubcoreMesh(core_axis_name="c", subcore_axis_name="s")

_i32 = jnp.int32


def _iota16():
    return lax.iota(_i32, 16)


@functools.partial(
    pl.kernel,
    mesh=_mesh,
    compiler_params=pltpu.CompilerParams(needs_layout_passes=False),
    out_type=(
        jax.ShapeDtypeStruct((_NW * _CELL,), _i32),
        jax.ShapeDtypeStruct((_NW * _CELL,), _i32),
        jax.ShapeDtypeStruct((_NW * _CELL,), _i32),
        jax.ShapeDtypeStruct((_NW * _NW,), _i32),
        jax.ShapeDtypeStruct((_NW * _NW,), _i32),
        jax.ShapeDtypeStruct((_NW * _NW,), _i32),
    ),
    scratch_types=[
        pltpu.VMEM((_B // _NW,), _i32),
        pltpu.VMEM((_CELL,), _i32),
        pltpu.VMEM((_NW,), _i32),
    ],
)
def _sc_route(users, pos_items, neg_items, ru, rp, rn, cru, crp, crn,
              idxb, cells, cnts):
    wid = lax.axis_index("s") * 2 + lax.axis_index("c")
    nsl = _B // _NW

    for arr, out_cells, out_cnts, shift, mask in (
            (pos_items, rp, crp, 15, _RI - 1),
            (neg_items, rn, crn, 15, _RI - 1),
            (users, ru, cru, 12, _RU - 1)):
        pltpu.sync_copy(arr.at[pl.ds(wid * nsl, nsl)], idxb)
        cnts[pl.ds(0, 16)] = jnp.zeros((16,), _i32)
        cnts[pl.ds(16, 16)] = jnp.zeros((16,), _i32)

        def vreg_body(g, carry):
            e = idxb[pl.ds(g * 16, 16)]
            slot = wid * nsl + g * 16 + _iota16()
            d = e >> shift
            rank, last = plsc.scan_count(d, None)
            cb = plsc.load_gather(cnts, [d])
            pos_ = d * _CAP0 + jnp.minimum(cb + rank - 1, _CAP0 - 1)
            pay = (e & mask) * 16384 + slot
            plsc.store_scatter(cells, [pos_], pay)
            plsc.store_scatter(cnts, [d], cb + rank, mask=last)
            return carry

        lax.fori_loop(0, nsl // 16, vreg_body, 0)
        pltpu.sync_copy(cells, out_cells.at[pl.ds(wid * _CELL, _CELL)])
        pltpu.sync_copy(cnts, out_cnts.at[pl.ds(wid * _NW, _NW)])


@functools.partial(
    pl.kernel,
    mesh=_mesh,
    compiler_params=pltpu.CompilerParams(needs_layout_passes=False),
    out_type=(
        jax.ShapeDtypeStruct(((_B + _NW) * _D,), jnp.float32),
        jax.ShapeDtypeStruct(((_B + _NW) * _D,), jnp.float32),
        jax.ShapeDtypeStruct(((_B + _NW) * _D,), jnp.float32),
    ),
    scratch_types=[
        pltpu.VMEM((_CELL,), _i32),
        pltpu.VMEM((_NW * _NW,), _i32),
        pltpu.VMEM((64,), _i32),
        pltpu.VMEM((64,), _i32),
        pltpu.VMEM((16,), _i32),
        pltpu.VMEM((_NB_I * _CAP_I,), _i32),
        pltpu.VMEM((_NB_I * _CAP_I,), _i32),
        pltpu.VMEM((_NB_U * _CAP_U,), _i32),
        pltpu.VMEM((_D, _SW), jnp.float32),
        pltpu.VMEM((_D, _SW), jnp.float32),
        pltpu.VMEM((256, _D), jnp.float32),
        pltpu.VMEM((16 * _D,), jnp.float32),
        pltpu.SemaphoreType.DMA,
        pltpu.SemaphoreType.DMA,
        pltpu.SemaphoreType.DMA,
    ],
)
def _sc_gather(users_t, items_t, ru, rp, rn, cru, crp, crn,
               gu, gp, gn,
               inc, rcnt, cp, cn, cu, bp, bn, bu,
               slab_a, slab_b, pack, drain, sem_a, sem_b, sem_s):
    wid = lax.axis_index("s") * 2 + lax.axis_index("c")
    it_lo = wid * _RI
    us_lo = wid * _RU

    z16 = jnp.zeros((16,), _i32)
    for q in range(4):
        cp[pl.ds(q * 16, 16)] = z16
        cn[pl.ds(q * 16, 16)] = z16
    cu[pl.ds(0, 16)] = z16

    def rebucket(cells_hbm, cnts_hbm, cnt_ref, buf_ref, cap):
        pltpu.sync_copy(cnts_hbm, rcnt)
        copies = []
        for s in range(_NW):
            copies.append(pltpu.async_copy(
                cells_hbm.at[pl.ds(s * _CELL + wid * _CAP0, _CAP0)],
                inc.at[pl.ds(s * _CAP0, _CAP0)], sem_a))
        for c in copies:
            c.wait()

        def src_body(s, carry):
            kv = plsc.load_gather(rcnt, [jnp.zeros((16,), _i32) + (s * _NW + wid)])
            kc = jnp.minimum(kv[0], _CAP0)

            def grp(g, carry2):
                pay = inc[pl.ds(s * _CAP0 + g * 16, 16)]
                m = (g * 16 + _iota16()) < kc
                el = pay >> 14
                b = jnp.where(m, el >> 9, 0)
                rank, last = plsc.scan_count(b, m)
                cb = plsc.load_gather(cnt_ref, [b], mask=m)
                pos_ = b * cap + jnp.minimum(cb + rank - 1, cap - 1)
                plsc.store_scatter(buf_ref, [pos_], pay, mask=m)
                plsc.store_scatter(cnt_ref, [b], cb + rank, mask=m & last)
                return carry2

            lax.fori_loop(0, (kc + 15) // 16, grp, 0)
            return carry

        lax.fori_loop(0, _NW, src_body, 0)

    rebucket(rp, crp, cp, bp, _CAP_I)
    rebucket(rn, crn, cn, bn, _CAP_I)
    rebucket(ru, cru, cu, bu, _CAP_U)

    def slab_start(lo_ent, s, pad_end):
        return jnp.minimum(lo_ent + s * _SW, pad_end - _SW)

    def extract(slab, start_rel, cnt_ref, buf_ref, s, cap, out_flat, issued0):
        kv = plsc.load_gather(cnt_ref, [jnp.full((16,), s, _i32)])
        k = jnp.minimum(kv[0], cap)

        def group_body(g, issued):
            @pl.when(issued >= 15)
            def _():
                pltpu.make_async_copy(out_flat.at[pl.ds(0, 16 * _D)], drain,
                                      sem_s).wait()

            ring = (issued & 15) * 16
            pay = buf_ref[pl.ds(s * cap + g * 16, 16)]
            valid = (g * 16 + _iota16()) < k
            el = pay >> 14
            lane = jnp.where(valid, el - start_rel, 0)
            slot = jnp.where(valid, pay & 16383, _B + wid)
            rows = ring + _iota16()
            for f in range(_D):
                vals = plsc.load_gather(slab, [jnp.full((16,), f, _i32), lane],
                                        mask=valid)
                plsc.store_scatter(pack, [rows, jnp.full((16,), f, _i32)],
                                   vals, mask=valid)
            for j in range(16):
                sl_ = slot[j]
                pltpu.async_copy(
                    pack.at[ring + j],
                    out_flat.at[pl.ds(pl.multiple_of(sl_ * _D, _D), _D)],
                    sem_s)
            return issued + 1

        return lax.fori_loop(0, (k + 15) // 16, group_body, issued0)

    def fire(table, lo_ent, s, pad_end, slab, sem):
        off = pl.multiple_of(slab_start(lo_ent, s, pad_end), 128)
        pltpu.async_copy(table.at[:, pl.ds(off, _SW)], slab, sem)

    def wait_slab(table, slab, sem):
        pltpu.make_async_copy(table.at[:, pl.ds(0, _SW)], slab, sem).wait()

    def run_table(table, lo_ent, n_slab, pad_end, jobs, issued0):
        fire(table, lo_ent, 0, pad_end, slab_a, sem_a)

        def slab_body(s, total):
            def on(slab, sem, other_slab, other_sem):
                res = total

                @pl.when(s + 1 < n_slab)
                def _():
                    fire(table, lo_ent, s + 1, pad_end, other_slab, other_sem)

                wait_slab(table, slab, sem)
                start_rel = slab_start(lo_ent, s, pad_end) - lo_ent
                for cnt_ref, buf_ref, cap, out_flat in jobs:
                    res = extract(slab, start_rel, cnt_ref, buf_ref, s, cap,
                                  out_flat, res)
                return res

            return lax.cond(s % 2 == 0,
                            lambda: on(slab_a, sem_a, slab_b, sem_b),
                            lambda: on(slab_b, sem_b, slab_a, sem_a))

        return lax.fori_loop(0, n_slab, slab_body, issued0)

    issued = run_table(items_t, it_lo, _NB_I, _PAD_I,
                       [(cp, bp, _CAP_I, gp), (cn, bn, _CAP_I, gn)],
                       jnp.int32(0))
    issued = run_table(users_t, us_lo, _NB_U, _PAD_U,
                       [(cu, bu, _CAP_U, gu)], issued)

    def drain_body(i, carry):
        pltpu.make_async_copy(gu.at[pl.ds(0, 16 * _D)], drain, sem_s).wait()
        return carry

    lax.fori_loop(0, jnp.minimum(issued, 15), drain_body, 0)


@functools.partial(
    pl.kernel,
    mesh=_mesh,
    compiler_params=pltpu.CompilerParams(needs_layout_passes=False),
    out_type=jax.ShapeDtypeStruct((_B, 16), jnp.float32),
    scratch_types=[
        pltpu.VMEM((128 * _D,), jnp.float32),
        pltpu.VMEM((128 * _D,), jnp.float32),
        pltpu.VMEM((128 * _D,), jnp.float32),
        pltpu.VMEM((_BPW, 16), jnp.float32),
        pltpu.SemaphoreType.DMA,
    ],
)
def _sc_scores(gu, gp, gn, out_hbm, ub, pb, nb, part, sem):
    wid = lax.axis_index("s") * 2 + lax.axis_index("c")
    base = wid * _BPW

    for h in range(_BPW // 128):
        off = (base + h * 128) * _D
        c1 = pltpu.async_copy(gu.at[pl.ds(off, 128 * _D)], ub, sem)
        c2 = pltpu.async_copy(gp.at[pl.ds(off, 128 * _D)], pb, sem)
        c3 = pltpu.async_copy(gn.at[pl.ds(off, 128 * _D)], nb, sem)
        c1.wait(); c2.wait(); c3.wait()

        def row_body(i, carry):
            t = jnp.zeros((16,), jnp.float32)
            for c in range(_D // 16):
                sl = pl.ds(i * _D + c * 16, 16)
                t = t + ub[sl] * (pb[sl] - nb[sl])
            part[h * 128 + i, :] = t
            return carry

        lax.fori_loop(0, 128, row_body, 0)

    pltpu.sync_copy(part, out_hbm.at[pl.ds(base, _BPW)])


def _tc_loss_body(part_ref, cu_ref, cp_ref, cn_ref, out_ref):
    diff = jnp.sum(part_ref[...], axis=1)
    ls = jnp.minimum(diff, 0.0) - jnp.log1p(jnp.exp(-jnp.abs(diff)))
    loss = -jnp.sum(ls) / _B
    reg = 0.5 * (jnp.sum(cu_ref[...] * cu_ref[...])
                 + jnp.sum(cp_ref[...] * cp_ref[...])
                 + jnp.sum(cn_ref[...] * cn_ref[...])) / _B
    out_ref[...] = jnp.broadcast_to(loss + _DECAY * reg, (1, 1))


def kernel(users_emb, items_emb, users, pos_items, neg_items,
           current_user_emb, current_pos_item_emb, current_neg_item_emb):
    ru, rp, rn, cru, crp, crn = _sc_route(users.astype(_i32),
                                          pos_items.astype(_i32),
                                          neg_items.astype(_i32))
    gu, gp, gn = _sc_gather(users_emb.T, items_emb.T,
                            ru, rp, rn, cru, crp, crn)
    part = _sc_scores(gu, gp, gn)
    out = pl.pallas_call(
        _tc_loss_body,
        out_shape=jax.ShapeDtypeStruct((1, 1), jnp.float32),
    )(part, current_user_emb.T, current_pos_item_emb.T, current_neg_item_emb.T)
    return out[0, 0]
